# baseline (speedup 1.0000x reference)
import functools

import jax
import jax.numpy as jnp
from jax import lax
from jax.experimental import pallas as pl
from jax.experimental.pallas import tpu as pltpu
from jax.experimental.pallas import tpu_sc as plsc

PAD = 100000
D = 128
L = 200
LP = 208
UNIT = 26
UPS = LP // UNIT
BATCH = 4096
NOUT = 100
NCORES = 2
NSUB = 16
NW = NCORES * NSUB
RPW = BATCH // NW
UNITS = UPS * RPW
NSLOTS = 16
LANES = 16
NCH = D // LANES


def _sc_rowsum(xh, table):
    mesh = plsc.VectorSubcoreMesh(
        core_axis_name="c", subcore_axis_name="s",
        num_cores=NCORES, num_subcores=NSUB)

    @functools.partial(
        pl.kernel,
        out_type=jax.ShapeDtypeStruct((BATCH, D), jnp.float32),
        mesh=mesh,
        compiler_params=pltpu.CompilerParams(
            use_tc_tiling_on_sc=False, needs_layout_passes=False),
        scratch_types=[
            pltpu.VMEM((UNITS, UNIT), jnp.int32),
            pltpu.VMEM((NSLOTS, UNIT, D), jnp.float32),
            pltpu.VMEM((RPW, D), jnp.float32),
            [pltpu.SemaphoreType.DMA] * NSLOTS,
        ],
    )
    def k(x_hbm, table_hbm, out_hbm, idx_v, rows_v, out_v, sems):
        wid = lax.axis_index("s") * NCORES + lax.axis_index("c")
        base = wid * UNITS

        pltpu.sync_copy(x_hbm.at[pl.ds(base, UNITS)], idx_v)

        def gather(slot, u):
            return pltpu.make_async_copy(
                table_hbm.at[idx_v.at[u]], rows_v.at[slot], sems[slot])

        for s in range(NSLOTS):
            gather(s, s).start()

        def accum_unit(slot, accs):
            def body(t, accs):
                return tuple(
                    accs[c] + rows_v[slot, t, pl.ds(c * LANES, LANES)]
                    for c in range(NCH))
            return plsc.parallel_loop(0, UNIT, carry=accs, unroll=2)(body)

        def loop_body(j, carry):
            for s in range(NSLOTS):
                u = NSLOTS * j + s
                gather(s, u).wait()
                if s % UPS == 0:
                    accs = tuple(
                        jnp.zeros((LANES,), jnp.float32) for _ in range(NCH))
                accs = accum_unit(s, accs)
                if s % UPS == UPS - 1:
                    row = (NSLOTS // UPS) * j + s // UPS
                    for c in range(NCH):
                        out_v[row, pl.ds(c * LANES, LANES)] = accs[c]

                @pl.when(u + NSLOTS < UNITS)
                def _():
                    gather(s, u + NSLOTS).start()
            return carry

        lax.fori_loop(0, UNITS // NSLOTS, loop_body, 0)
        pltpu.sync_copy(out_v, out_hbm.at[pl.ds(wid * RPW, RPW)])

    return k(xh, table)


def _tc_head(x, summed, W, b):
    blk = 512
    grid = BATCH // blk

    def body(x_ref, s_ref, w_ref, b_ref, o_ref):
        cnt = jnp.sum((x_ref[...] != PAD).astype(jnp.float32),
                      axis=1, keepdims=True)
        mean = s_ref[...] / jnp.maximum(cnt, 1e-6)
        o_ref[...] = jnp.dot(mean, w_ref[...],
                             preferred_element_type=jnp.float32) + b_ref[...]

    return pl.pallas_call(
        body,
        grid=(grid,),
        in_specs=[
            pl.BlockSpec((blk, L), lambda i: (i, 0)),
            pl.BlockSpec((blk, D), lambda i: (i, 0)),
            pl.BlockSpec((D, NOUT), lambda i: (0, 0)),
            pl.BlockSpec((1, NOUT), lambda i: (0, 0)),
        ],
        out_specs=pl.BlockSpec((blk, NOUT), lambda i: (i, 0)),
        out_shape=jax.ShapeDtypeStruct((BATCH, NOUT), jnp.float32),
    )(x, summed, W, b.reshape(1, NOUT))


def kernel(x, table, W, b):
    xp = jnp.pad(x, ((0, 0), (0, LP - L)), constant_values=PAD)
    xh = xp.reshape(UPS * BATCH, UNIT)
    summed = _sc_rowsum(xh, table)
    return _tc_head(x, summed, W, b)


# speedup vs baseline: 1.6209x; 1.6209x over previous
import functools

import jax
import jax.numpy as jnp
import numpy as np
from jax import lax
from jax.experimental import pallas as pl
from jax.experimental.pallas import tpu as pltpu
from jax.experimental.pallas import tpu_sc as plsc

PAD = 100000
D = 128
L = 200
LP = 208
UNIT = 26
UPS = LP // UNIT
BATCH = 4096
NOUT = 100
NCORES = 2
NSUB = 16
NW = NCORES * NSUB
RPW = BATCH // NW
UNITS = UPS * RPW
NSLOTS = 16
LANES = 16
NCH = D // LANES
NBH = D // 32

_PERM = np.concatenate(
    [np.concatenate([np.arange(32 * g, 32 * (g + 1), 2),
                     np.arange(32 * g + 1, 32 * (g + 1), 2)])
     for g in range(NBH)])


def _sc_rowsum(xh, table):
    mesh = plsc.VectorSubcoreMesh(
        core_axis_name="c", subcore_axis_name="s",
        num_cores=NCORES, num_subcores=NSUB)

    @functools.partial(
        pl.kernel,
        out_type=jax.ShapeDtypeStruct((BATCH, D), jnp.float32),
        mesh=mesh,
        compiler_params=pltpu.CompilerParams(
            use_tc_tiling_on_sc=False, needs_layout_passes=False),
        scratch_types=[
            pltpu.VMEM((UNITS, UNIT), jnp.int32),
            pltpu.VMEM((NSLOTS, UNIT, D), jnp.bfloat16),
            pltpu.VMEM((RPW, D), jnp.float32),
            [pltpu.SemaphoreType.DMA] * NSLOTS,
        ],
    )
    def k(x_hbm, table_hbm, out_hbm, idx_v, rows_v, out_v, sems):
        wid = lax.axis_index("s") * NCORES + lax.axis_index("c")
        base = wid * UNITS

        pltpu.sync_copy(x_hbm.at[pl.ds(base, UNITS)], idx_v)

        def gather(slot, u):
            return pltpu.make_async_copy(
                table_hbm.at[idx_v.at[u]], rows_v.at[slot], sems[slot])

        for s in range(NSLOTS):
            gather(s, s).start()

        def accum_unit(slot, accs):
            def body(t, accs):
                new = []
                for c in range(NBH):
                    v = rows_v[slot, t, pl.ds(c * 32, 32)]
                    lo, hi = plsc.unpack(
                        v, format=plsc.PackFormat.INTERLEAVED,
                        preferred_element_type=jnp.float32)
                    new.append(accs[2 * c] + lo)
                    new.append(accs[2 * c + 1] + hi)
                return tuple(new)
            return plsc.parallel_loop(0, UNIT, carry=accs, unroll=2)(body)

        def loop_body(j, carry):
            for s in range(NSLOTS):
                u = NSLOTS * j + s
                gather(s, u).wait()
                if s % UPS == 0:
                    accs = tuple(
                        jnp.zeros((LANES,), jnp.float32) for _ in range(NCH))
                accs = accum_unit(s, accs)
                if s % UPS == UPS - 1:
                    row = (NSLOTS // UPS) * j + s // UPS
                    for c in range(NCH):
                        out_v[row, pl.ds(c * LANES, LANES)] = accs[c]

                @pl.when(u + NSLOTS < UNITS)
                def _():
                    gather(s, u + NSLOTS).start()
            return carry

        lax.fori_loop(0, UNITS // NSLOTS, loop_body, 0)
        pltpu.sync_copy(out_v, out_hbm.at[pl.ds(wid * RPW, RPW)])

    return k(xh, table)


def _tc_head(x, summed, W, b):
    blk = 512
    grid = BATCH // blk

    def body(x_ref, s_ref, w_ref, b_ref, o_ref):
        cnt = jnp.sum((x_ref[...] != PAD).astype(jnp.float32),
                      axis=1, keepdims=True)
        mean = s_ref[...] / jnp.maximum(cnt, 1e-6)
        o_ref[...] = jnp.dot(mean, w_ref[...],
                             preferred_element_type=jnp.float32) + b_ref[...]

    return pl.pallas_call(
        body,
        grid=(grid,),
        in_specs=[
            pl.BlockSpec((blk, L), lambda i: (i, 0)),
            pl.BlockSpec((blk, D), lambda i: (i, 0)),
            pl.BlockSpec((D, NOUT), lambda i: (0, 0)),
            pl.BlockSpec((1, NOUT), lambda i: (0, 0)),
        ],
        out_specs=pl.BlockSpec((blk, NOUT), lambda i: (i, 0)),
        out_shape=jax.ShapeDtypeStruct((BATCH, NOUT), jnp.float32),
    )(x, summed, W, b.reshape(1, NOUT))


def kernel(x, table, W, b):
    xp = jnp.pad(x, ((0, 0), (0, LP - L)), constant_values=PAD)
    xh = xp.reshape(UPS * BATCH, UNIT)
    summed = _sc_rowsum(xh, table.astype(jnp.bfloat16))
    return _tc_head(x, summed, W[_PERM, :], b)
